# Initial kernel scaffold; baseline (speedup 1.0000x reference)
import jax
import jax.numpy as jnp
from jax.experimental import pallas as pl


def kernel(emb, edges, w_weight, w_bias, a_weight, a_bias, trans_weight, trans_bias, ln_gamma, ln_beta):
    raise NotImplementedError("write your pallas kernel here")


# speedup vs baseline: 3.7572x; 3.7572x over previous
import dataclasses
import functools

import jax
import jax.numpy as jnp
from jax import lax
from jax.experimental import pallas as pl
from jax.experimental.pallas import tpu as pltpu
from jax.experimental.pallas import tpu_sc as plsc

N = 10000
E = 160000
D = 256
DH = 128
NS = 16
K = 64
EPAD = 160768
EPS = EPAD // NS
NBLK = EPS // K
STRIPE = 624
TAIL = N - NS * STRIPE
NP = 10112
RST = 640


def _s12_body(emb_ref, w_ref, wb_ref, a_ref, ab_ref, s1_ref, s2_ref):
    a1 = a_ref[:, 0:D]
    a2 = a_ref[:, D : 2 * D]
    dn_av = (((1,), (0,)), ((), ()))
    c1 = lax.dot_general(a1, w_ref[...], dn_av,
                         preferred_element_type=jnp.float32)
    c2 = lax.dot_general(a2, w_ref[...], dn_av,
                         preferred_element_type=jnp.float32)
    b1 = jnp.sum(wb_ref[...] * a1[0, :])
    b2 = jnp.sum(wb_ref[...] * a2[0, :]) + ab_ref[0]
    dn_ec = (((1,), (1,)), ((), ()))
    s1 = lax.dot_general(c1, emb_ref[...], dn_ec,
                         preferred_element_type=jnp.float32)
    s2 = lax.dot_general(c2, emb_ref[...], dn_ec,
                         preferred_element_type=jnp.float32)
    s1_ref[...] = s1[0] + b1
    s2_ref[...] = s2[0] + b2


def _s12_call(emb, w_weight, w_bias, a_weight, a_bias):
    return pl.pallas_call(
        _s12_body,
        out_shape=[jax.ShapeDtypeStruct((N,), jnp.float32),
                   jax.ShapeDtypeStruct((N,), jnp.float32)],
    )(emb, w_weight, w_bias, a_weight, a_bias)


_sc_mesh = plsc.VectorSubcoreMesh(core_axis_name="c", subcore_axis_name="s")

_sc_params = pltpu.CompilerParams()
if "needs_layout_passes" in pltpu.CompilerParams.__dataclass_fields__:
    _sc_params = dataclasses.replace(_sc_params, needs_layout_passes=False)


@functools.partial(
    pl.kernel,
    out_type=[
        jax.ShapeDtypeStruct((N, DH), jnp.float32),
        jax.ShapeDtypeStruct((N, DH), jnp.float32),
        jax.ShapeDtypeStruct((NP,), jnp.float32),
    ],
    mesh=_sc_mesh,
    scratch_types=[
        pltpu.VMEM((N,), jnp.float32),
        pltpu.VMEM((N,), jnp.float32),
        pltpu.VMEM((NP,), jnp.float32),
        pltpu.VMEM((K,), jnp.int32),
        pltpu.VMEM((K,), jnp.int32),
        pltpu.VMEM((K,), jnp.int32),
        pltpu.VMEM((K,), jnp.float32),
        pltpu.VMEM((K, DH), jnp.float32),
        pltpu.VMEM_SHARED((N, DH), jnp.float32),
        pltpu.VMEM_SHARED((NS, NP), jnp.float32),
    ],
    compiler_params=_sc_params,
)
def _sc_agg(embi_hbm, srcp_hbm, dstp_hbm, s1_hbm, s2_hbm,
            aggl_hbm, aggr_hbm, asum_hbm,
            s1_v, s2_v, asum_v, src_v, dst_v, gidx_v, attn_v, rows_v,
            acc, stage):
    cid = lax.axis_index("c")
    sid = lax.axis_index("s")

    zero16 = jnp.zeros((16,), jnp.float32)

    @pl.loop(0, K)
    def _(r):
        for ch in range(DH // 16):
            rows_v[r, pl.ds(ch * 16, 16)] = zero16

    @pl.loop(0, NP // 16)
    def _(i):
        asum_v[pl.ds(i * 16, 16)] = zero16

    for j in range(STRIPE // K):
        pltpu.sync_copy(rows_v, acc.at[pl.ds(sid * STRIPE + j * K, K)])
    rem = STRIPE - (STRIPE // K) * K
    if rem:
        pltpu.sync_copy(rows_v.at[pl.ds(0, rem)],
                        acc.at[pl.ds(sid * STRIPE + (STRIPE // K) * K, rem)])

    @pl.when(sid == NS - 1)
    def _():
        pltpu.sync_copy(rows_v.at[pl.ds(0, TAIL)],
                        acc.at[pl.ds(NS * STRIPE, TAIL)])

    pltpu.sync_copy(s1_hbm, s1_v)
    pltpu.sync_copy(s2_hbm, s2_v)

    plsc.subcore_barrier()

    base0 = sid * EPS

    @pl.loop(0, NBLK)
    def _(g):
        base = base0 + g * K
        pltpu.sync_copy(srcp_hbm.at[pl.ds(base, K)], src_v)
        pltpu.sync_copy(dstp_hbm.at[pl.ds(base, K)], dst_v)

        for t in range(K // 16):
            sl = pl.ds(t * 16, 16)
            s16 = src_v[sl]
            d16 = dst_v[sl]
            gidx_v[sl] = d16 * 2 + cid
            lg = plsc.load_gather(s1_v, [s16]) + plsc.load_gather(s2_v, [d16])
            lr = jnp.where(lg > 0.0, lg, 0.2 * lg)
            at = jnp.exp(lr)
            glob = base + t * 16 + lax.iota(jnp.int32, 16)
            at = jnp.where(glob < E, at, 0.0)
            attn_v[sl] = at
            plsc.addupdate_scatter(asum_v, [s16], at)

        pltpu.sync_copy(embi_hbm.at[gidx_v], rows_v)

        @pl.loop(0, K)
        def _(r):
            sp = plsc.load_gather(attn_v, [jnp.full((16,), r, jnp.int32)])
            for ch in range(DH // 16):
                cs = pl.ds(ch * 16, 16)
                rows_v[r, cs] = rows_v[r, cs] * sp

        pltpu.sync_copy(rows_v, acc.at[src_v], add=True)

    pltpu.sync_copy(asum_v, stage.at[sid])

    plsc.subcore_barrier()

    rs = pl.ds(sid * STRIPE, STRIPE)
    tl = pl.ds(NS * STRIPE, TAIL)
    last = sid == NS - 1

    @pl.when(cid == 0)
    def _():
        pltpu.sync_copy(acc.at[rs], aggl_hbm.at[rs])

        @pl.when(last)
        def _():
            pltpu.sync_copy(acc.at[tl], aggl_hbm.at[tl])

    @pl.when(cid == 1)
    def _():
        pltpu.sync_copy(acc.at[rs], aggr_hbm.at[rs])

        @pl.when(last)
        def _():
            pltpu.sync_copy(acc.at[tl], aggr_hbm.at[tl])

    @pl.when(cid == 0)
    def _():
        nchunk = RST // DH
        for j in range(nchunk):
            n0 = sid * RST + j * DH

            @pl.when(n0 + DH <= NP)
            def _():
                pltpu.sync_copy(stage.at[:, pl.ds(n0, DH)],
                                rows_v.at[pl.ds(0, NS)])
                for w in range(1, NS):
                    for ch in range(DH // 16):
                        cs = pl.ds(ch * 16, 16)
                        rows_v[0, cs] = rows_v[0, cs] + rows_v[w, cs]
                pltpu.sync_copy(rows_v.at[0], asum_hbm.at[pl.ds(n0, DH)])


_RB = 2000


def _post_body(emb_ref, aggl_ref, aggr_ref, asum_ref, tw_ref, tb_ref,
               g_ref, b_ref, out_ref):
    inv = 1.0 / (asum_ref[...] + 1e-20)
    dn = (((1,), (1,)), ((), ()))
    h = lax.dot_general(emb_ref[...], tw_ref[:, 0:D], dn,
                        preferred_element_type=jnp.float32)
    h = h + lax.dot_general(aggl_ref[...] * inv, tw_ref[:, D : D + DH], dn,
                            preferred_element_type=jnp.float32)
    h = h + lax.dot_general(aggr_ref[...] * inv, tw_ref[:, D + DH : 2 * D],
                            dn, preferred_element_type=jnp.float32)
    h = h + tb_ref[...][None, :]
    mean = jnp.mean(h, axis=1, keepdims=True)
    var = jnp.mean((h - mean) ** 2, axis=1, keepdims=True)
    out_ref[...] = ((h - mean) * lax.rsqrt(var + 1e-5) * g_ref[...][None, :]
                    + b_ref[...][None, :])


def _post_call(emb, aggl, aggr, asum, trans_weight, trans_bias,
               ln_gamma, ln_beta):
    nblk = N // _RB
    return pl.pallas_call(
        _post_body,
        grid=(nblk,),
        in_specs=[
            pl.BlockSpec((_RB, D), lambda i: (i, 0)),
            pl.BlockSpec((_RB, DH), lambda i: (i, 0)),
            pl.BlockSpec((_RB, DH), lambda i: (i, 0)),
            pl.BlockSpec((_RB, 1), lambda i: (i, 0)),
            pl.BlockSpec((D, 2 * D), lambda i: (0, 0)),
            pl.BlockSpec((D,), lambda i: (0,)),
            pl.BlockSpec((D,), lambda i: (0,)),
            pl.BlockSpec((D,), lambda i: (0,)),
        ],
        out_specs=pl.BlockSpec((_RB, D), lambda i: (i, 0)),
        out_shape=jax.ShapeDtypeStruct((N, D), jnp.float32),
    )(emb, aggl, aggr, asum[:N, None], trans_weight, trans_bias,
      ln_gamma, ln_beta)


def kernel(emb, edges, w_weight, w_bias, a_weight, a_bias,
           trans_weight, trans_bias, ln_gamma, ln_beta):
    src = edges[0]
    dst = edges[1]
    pad = EPAD - E
    srcp = jnp.concatenate([src, jnp.zeros((pad,), jnp.int32)])
    dstp = jnp.concatenate([dst, jnp.zeros((pad,), jnp.int32)])
    embi = emb.reshape(N, 2, DH).reshape(2 * N, DH)

    s1, s2 = _s12_call(emb, w_weight, w_bias, a_weight, a_bias)
    aggl, aggr, asum = _sc_agg(embi, srcp, dstp, s1, s2)
    return _post_call(emb, aggl, aggr, asum, trans_weight, trans_bias,
                      ln_gamma, ln_beta)


# speedup vs baseline: 6.0695x; 1.6155x over previous
import dataclasses
import functools

import jax
import jax.numpy as jnp
from jax import lax
from jax.experimental import pallas as pl
from jax.experimental.pallas import tpu as pltpu
from jax.experimental.pallas import tpu_sc as plsc

N = 10000
E = 160000
D = 256
DH = 128
NS = 16
K = 64
NBLK = 158
EPS = NBLK * K
EPAD = NS * EPS
STRIPE = 624
TAIL = N - NS * STRIPE
NP = 10112
RST = 640


def _s12_body(emb_ref, w_ref, wb_ref, a_ref, ab_ref, s1_ref, s2_ref):
    a1 = a_ref[:, 0:D]
    a2 = a_ref[:, D : 2 * D]
    dn_av = (((1,), (0,)), ((), ()))
    c1 = lax.dot_general(a1, w_ref[...], dn_av,
                         preferred_element_type=jnp.float32)
    c2 = lax.dot_general(a2, w_ref[...], dn_av,
                         preferred_element_type=jnp.float32)
    b1 = jnp.sum(wb_ref[...] * a1[0, :])
    b2 = jnp.sum(wb_ref[...] * a2[0, :]) + ab_ref[0]
    dn_ec = (((1,), (1,)), ((), ()))
    s1 = lax.dot_general(c1, emb_ref[...], dn_ec,
                         preferred_element_type=jnp.float32)
    s2 = lax.dot_general(c2, emb_ref[...], dn_ec,
                         preferred_element_type=jnp.float32)
    s1_ref[...] = s1[0] + b1
    s2_ref[...] = s2[0] + b2


def _s12_call(emb, w_weight, w_bias, a_weight, a_bias):
    return pl.pallas_call(
        _s12_body,
        out_shape=[jax.ShapeDtypeStruct((N,), jnp.float32),
                   jax.ShapeDtypeStruct((N,), jnp.float32)],
    )(emb, w_weight, w_bias, a_weight, a_bias)


_sc_mesh = plsc.VectorSubcoreMesh(core_axis_name="c", subcore_axis_name="s")

_sc_params = pltpu.CompilerParams()
if "needs_layout_passes" in pltpu.CompilerParams.__dataclass_fields__:
    _sc_params = dataclasses.replace(_sc_params, needs_layout_passes=False)


@functools.partial(
    pl.kernel,
    out_type=[
        jax.ShapeDtypeStruct((N, DH), jnp.float32),
        jax.ShapeDtypeStruct((N, DH), jnp.float32),
        jax.ShapeDtypeStruct((NP,), jnp.float32),
        jax.ShapeDtypeStruct((NS, NP), jnp.float32),
    ],
    mesh=_sc_mesh,
    scratch_types=[
        pltpu.VMEM((N,), jnp.float32),
        pltpu.VMEM((N,), jnp.float32),
        pltpu.VMEM((NP,), jnp.float32),
        pltpu.VMEM((2 * K,), jnp.int32),
        pltpu.VMEM((2 * K,), jnp.int32),
        pltpu.VMEM((K,), jnp.int32),
        pltpu.VMEM((K,), jnp.int32),
        pltpu.VMEM((K,), jnp.int32),
        pltpu.VMEM((K,), jnp.int32),
        pltpu.VMEM((K,), jnp.float32),
        pltpu.VMEM((K,), jnp.float32),
        pltpu.VMEM((K, DH), jnp.float32),
        pltpu.VMEM((K, DH), jnp.float32),
        pltpu.SemaphoreType.DMA,
        pltpu.SemaphoreType.DMA,
        pltpu.SemaphoreType.DMA,
        pltpu.SemaphoreType.DMA,
        pltpu.SemaphoreType.DMA,
        pltpu.SemaphoreType.DMA,
        pltpu.VMEM_SHARED((N, DH), jnp.float32),
    ],
    compiler_params=_sc_params,
)
def _sc_agg(embi_hbm, idx_hbm, s1_hbm, s2_hbm,
            aggl_hbm, aggr_hbm, asum_hbm, stage_hbm,
            s1_v, s2_v, asum_v, idx0_v, idx1_v, gidx0_v, gidx1_v,
            srcc0_v, srcc1_v, attn0_v, attn1_v, rows0_v, rows1_v,
            isem0, isem1, gsem0, gsem1, csem0, csem1,
            acc):
    cid = lax.axis_index("c")
    sid = lax.axis_index("s")

    idx = (idx0_v, idx1_v)
    gidx = (gidx0_v, gidx1_v)
    srcc = (srcc0_v, srcc1_v)
    attn = (attn0_v, attn1_v)
    rows = (rows0_v, rows1_v)
    isem = (isem0, isem1)
    gsem = (gsem0, gsem1)
    csem = (csem0, csem1)

    zero16 = jnp.zeros((16,), jnp.float32)

    @pl.loop(0, K)
    def _(r):
        for ch in range(DH // 16):
            rows0_v[r, pl.ds(ch * 16, 16)] = zero16

    @pl.loop(0, NP // 16)
    def _(i):
        asum_v[pl.ds(i * 16, 16)] = zero16

    for j in range(STRIPE // K):
        pltpu.sync_copy(rows0_v, acc.at[pl.ds(sid * STRIPE + j * K, K)])
    rem = STRIPE - (STRIPE // K) * K
    if rem:
        pltpu.sync_copy(rows0_v.at[pl.ds(0, rem)],
                        acc.at[pl.ds(sid * STRIPE + (STRIPE // K) * K, rem)])

    @pl.when(sid == NS - 1)
    def _():
        pltpu.sync_copy(rows0_v.at[pl.ds(0, TAIL)],
                        acc.at[pl.ds(NS * STRIPE, TAIL)])

    pltpu.sync_copy(s1_hbm, s1_v)
    pltpu.sync_copy(s2_hbm, s2_v)

    plsc.subcore_barrier()

    base0 = sid * EPS

    def start_idx(g, slot):
        pltpu.async_copy(
            idx_hbm.at[pl.ds((sid * NBLK + g) * 2 * K, 2 * K)],
            idx[slot], isem[slot])

    def wait_idx(slot):
        pltpu.make_async_copy(idx_hbm.at[pl.ds(0, 2 * K)], idx[slot],
                              isem[slot]).wait()

    def compute_attn(g, slot):
        for t in range(K // 16):
            sl = pl.ds(t * 16, 16)
            s16 = idx[slot][sl]
            d16 = idx[slot][pl.ds(K + t * 16, 16)]
            gidx[slot][sl] = d16 * 2 + cid
            srcc[slot][sl] = s16
            lg = (plsc.load_gather(s1_v, [s16])
                  + plsc.load_gather(s2_v, [d16]))
            lr = jnp.where(lg > 0.0, lg, 0.2 * lg)
            at = jnp.exp(lr)
            glob = base0 + g * K + t * 16 + lax.iota(jnp.int32, 16)
            at = jnp.where(glob < E, at, 0.0)
            attn[slot][sl] = at
            plsc.addupdate_scatter(asum_v, [s16], at)

    def start_gather(slot):
        pltpu.async_copy(embi_hbm.at[gidx[slot]], rows[slot], gsem[slot])

    def wait_gather(slot):
        pltpu.make_async_copy(embi_hbm.at[gidx[slot]], rows[slot],
                              gsem[slot]).wait()

    def scale_rows(slot):
        rv = rows[slot]
        av = attn[slot]

        @pl.loop(0, K)
        def _(r):
            sp = plsc.load_gather(av, [jnp.full((16,), r, jnp.int32)])
            for ch in range(DH // 16):
                cs = pl.ds(ch * 16, 16)
                rv[r, cs] = rv[r, cs] * sp

    def start_scatter(slot):
        pltpu.async_copy(rows[slot], acc.at[srcc[slot]], csem[slot],
                         add=True)

    def wait_scatter(slot):
        pltpu.make_async_copy(rows[slot], acc.at[srcc[slot]],
                              csem[slot]).wait()

    def steady(g, slot):
        nslot = 1 - slot
        wait_scatter(nslot)
        wait_idx(nslot)
        compute_attn(g + 1, nslot)
        start_idx(g + 3, nslot)
        start_gather(nslot)
        wait_gather(slot)
        scale_rows(slot)
        start_scatter(slot)

    start_idx(0, 0)
    start_idx(1, 1)
    wait_idx(0)
    compute_attn(0, 0)
    start_idx(2, 0)
    start_gather(0)
    wait_idx(1)
    compute_attn(1, 1)
    start_idx(3, 1)
    start_gather(1)
    wait_gather(0)
    scale_rows(0)
    start_scatter(0)

    @pl.loop(0, (NBLK - 2) // 2)
    def _(t):
        steady(2 * t + 1, 1)
        steady(2 * t + 2, 0)

    wait_gather(1)
    scale_rows(1)
    start_scatter(1)
    wait_scatter(0)
    wait_scatter(1)
    wait_idx(0)
    wait_idx(1)

    @pl.when(cid == 0)
    def _():
        pltpu.sync_copy(asum_v, stage_hbm.at[sid])

    plsc.subcore_barrier()

    rs = pl.ds(sid * STRIPE, STRIPE)
    tl = pl.ds(NS * STRIPE, TAIL)
    last = sid == NS - 1

    @pl.when(cid == 0)
    def _():
        pltpu.sync_copy(acc.at[rs], aggl_hbm.at[rs])

        @pl.when(last)
        def _():
            pltpu.sync_copy(acc.at[tl], aggl_hbm.at[tl])

    @pl.when(cid == 1)
    def _():
        pltpu.sync_copy(acc.at[rs], aggr_hbm.at[rs])

        @pl.when(last)
        def _():
            pltpu.sync_copy(acc.at[tl], aggr_hbm.at[tl])

    @pl.when(cid == 0)
    def _():
        for j in range(RST // DH):
            n0 = sid * RST + j * DH

            @pl.when(n0 + DH <= NP)
            def _():
                pltpu.sync_copy(stage_hbm.at[:, pl.ds(n0, DH)],
                                rows0_v.at[pl.ds(0, NS)])
                for w in range(1, NS):
                    for ch in range(DH // 16):
                        cs = pl.ds(ch * 16, 16)
                        rows0_v[0, cs] = rows0_v[0, cs] + rows0_v[w, cs]
                pltpu.sync_copy(rows0_v.at[0], asum_hbm.at[pl.ds(n0, DH)])


_RB = 2000


def _post_body(emb_ref, aggl_ref, aggr_ref, asum_ref, tw_ref, tb_ref,
               g_ref, b_ref, out_ref):
    inv = 1.0 / (asum_ref[...] + 1e-20)
    dn = (((1,), (1,)), ((), ()))
    h = lax.dot_general(emb_ref[...], tw_ref[:, 0:D], dn,
                        preferred_element_type=jnp.float32)
    h = h + lax.dot_general(aggl_ref[...] * inv, tw_ref[:, D : D + DH], dn,
                            preferred_element_type=jnp.float32)
    h = h + lax.dot_general(aggr_ref[...] * inv, tw_ref[:, D + DH : 2 * D],
                            dn, preferred_element_type=jnp.float32)
    h = h + tb_ref[...][None, :]
    mean = jnp.mean(h, axis=1, keepdims=True)
    var = jnp.mean((h - mean) ** 2, axis=1, keepdims=True)
    out_ref[...] = ((h - mean) * lax.rsqrt(var + 1e-5) * g_ref[...][None, :]
                    + b_ref[...][None, :])


def _post_call(emb, aggl, aggr, asum, trans_weight, trans_bias,
               ln_gamma, ln_beta):
    nblk = N // _RB
    return pl.pallas_call(
        _post_body,
        grid=(nblk,),
        in_specs=[
            pl.BlockSpec((_RB, D), lambda i: (i, 0)),
            pl.BlockSpec((_RB, DH), lambda i: (i, 0)),
            pl.BlockSpec((_RB, DH), lambda i: (i, 0)),
            pl.BlockSpec((_RB, 1), lambda i: (i, 0)),
            pl.BlockSpec((D, 2 * D), lambda i: (0, 0)),
            pl.BlockSpec((D,), lambda i: (0,)),
            pl.BlockSpec((D,), lambda i: (0,)),
            pl.BlockSpec((D,), lambda i: (0,)),
        ],
        out_specs=pl.BlockSpec((_RB, D), lambda i: (i, 0)),
        out_shape=jax.ShapeDtypeStruct((N, D), jnp.float32),
    )(emb, aggl, aggr, asum[:N, None], trans_weight, trans_bias,
      ln_gamma, ln_beta)


def kernel(emb, edges, w_weight, w_bias, a_weight, a_bias,
           trans_weight, trans_bias, ln_gamma, ln_beta):
    src = edges[0]
    dst = edges[1]
    pad = EPAD - E
    srcp = jnp.concatenate([src, jnp.zeros((pad,), jnp.int32)])
    dstp = jnp.concatenate([dst, jnp.zeros((pad,), jnp.int32)])
    idxc = jnp.stack([srcp.reshape(-1, K), dstp.reshape(-1, K)],
                     axis=1).reshape(-1)
    idxc = jnp.concatenate([idxc, jnp.zeros((2 * 2 * K,), jnp.int32)])
    embi = emb.reshape(N, 2, DH).reshape(2 * N, DH)

    s1, s2 = _s12_call(emb, w_weight, w_bias, a_weight, a_bias)
    aggl, aggr, asum, _ = _sc_agg(embi, idxc, s1, s2)
    return _post_call(emb, aggl, aggr, asum, trans_weight, trans_bias,
                      ln_gamma, ln_beta)


# speedup vs baseline: 6.1825x; 1.0186x over previous
import dataclasses
import functools

import jax
import jax.numpy as jnp
from jax import lax
from jax.experimental import pallas as pl
from jax.experimental.pallas import tpu as pltpu
from jax.experimental.pallas import tpu_sc as plsc

N = 10000
E = 160000
D = 256
DH = 128
NS = 16
K = 64
NBLK = 158
EPS = NBLK * K
EPAD = NS * EPS
STRIPE = 624
TAIL = N - NS * STRIPE
NP = 10112
RST = 640


def _s12_body(emb_ref, w_ref, wb_ref, a_ref, ab_ref, s1_ref, s2_ref):
    a1 = a_ref[:, 0:D]
    a2 = a_ref[:, D : 2 * D]
    dn_av = (((1,), (0,)), ((), ()))
    c1 = lax.dot_general(a1, w_ref[...], dn_av,
                         preferred_element_type=jnp.float32)
    c2 = lax.dot_general(a2, w_ref[...], dn_av,
                         preferred_element_type=jnp.float32)
    b1 = jnp.sum(wb_ref[...] * a1[0, :])
    b2 = jnp.sum(wb_ref[...] * a2[0, :]) + ab_ref[0]
    dn_ec = (((1,), (1,)), ((), ()))
    s1 = lax.dot_general(c1, emb_ref[...], dn_ec,
                         preferred_element_type=jnp.float32)
    s2 = lax.dot_general(c2, emb_ref[...], dn_ec,
                         preferred_element_type=jnp.float32)
    s1_ref[...] = s1[0] + b1
    s2_ref[...] = s2[0] + b2


def _s12_call(emb, w_weight, w_bias, a_weight, a_bias):
    return pl.pallas_call(
        _s12_body,
        out_shape=[jax.ShapeDtypeStruct((N,), jnp.float32),
                   jax.ShapeDtypeStruct((N,), jnp.float32)],
    )(emb, w_weight, w_bias, a_weight, a_bias)


_sc_mesh = plsc.VectorSubcoreMesh(core_axis_name="c", subcore_axis_name="s")

_sc_params = pltpu.CompilerParams()
if "needs_layout_passes" in pltpu.CompilerParams.__dataclass_fields__:
    _sc_params = dataclasses.replace(_sc_params, needs_layout_passes=False)


@functools.partial(
    pl.kernel,
    out_type=[
        jax.ShapeDtypeStruct((N, DH), jnp.float32),
        jax.ShapeDtypeStruct((N, DH), jnp.float32),
        jax.ShapeDtypeStruct((NP,), jnp.float32),
        jax.ShapeDtypeStruct((NS, NP), jnp.float32),
    ],
    mesh=_sc_mesh,
    scratch_types=[
        pltpu.VMEM((N,), jnp.float32),
        pltpu.VMEM((N,), jnp.float32),
        pltpu.VMEM((NP,), jnp.float32),
        pltpu.VMEM((2 * K,), jnp.int32),
        pltpu.VMEM((2 * K,), jnp.int32),
        pltpu.VMEM((K,), jnp.int32),
        pltpu.VMEM((K,), jnp.int32),
        pltpu.VMEM((K,), jnp.int32),
        pltpu.VMEM((K,), jnp.int32),
        pltpu.VMEM((K,), jnp.float32),
        pltpu.VMEM((K,), jnp.float32),
        pltpu.VMEM((K, DH), jnp.float32),
        pltpu.VMEM((K, DH), jnp.float32),
        pltpu.SemaphoreType.DMA,
        pltpu.SemaphoreType.DMA,
        pltpu.SemaphoreType.DMA,
        pltpu.SemaphoreType.DMA,
        pltpu.SemaphoreType.DMA,
        pltpu.SemaphoreType.DMA,
        pltpu.VMEM_SHARED((N, DH), jnp.float32),
    ],
    compiler_params=_sc_params,
)
def _sc_agg(embi_hbm, idx_hbm, s1_hbm, s2_hbm,
            aggl_hbm, aggr_hbm, asum_hbm, stage_hbm,
            s1_v, s2_v, asum_v, idx0_v, idx1_v, gidx0_v, gidx1_v,
            srcc0_v, srcc1_v, attn0_v, attn1_v, rows0_v, rows1_v,
            isem0, isem1, gsem0, gsem1, csem0, csem1,
            acc):
    cid = lax.axis_index("c")
    sid = lax.axis_index("s")

    idx = (idx0_v, idx1_v)
    gidx = (gidx0_v, gidx1_v)
    srcc = (srcc0_v, srcc1_v)
    attn = (attn0_v, attn1_v)
    rows = (rows0_v, rows1_v)
    isem = (isem0, isem1)
    gsem = (gsem0, gsem1)
    csem = (csem0, csem1)

    zero16 = jnp.zeros((16,), jnp.float32)

    @pl.loop(0, K)
    def _(r):
        for ch in range(DH // 16):
            rows0_v[r, pl.ds(ch * 16, 16)] = zero16

    @pl.loop(0, NP // 16)
    def _(i):
        asum_v[pl.ds(i * 16, 16)] = zero16

    for j in range(STRIPE // K):
        pltpu.sync_copy(rows0_v, acc.at[pl.ds(sid * STRIPE + j * K, K)])
    rem = STRIPE - (STRIPE // K) * K
    if rem:
        pltpu.sync_copy(rows0_v.at[pl.ds(0, rem)],
                        acc.at[pl.ds(sid * STRIPE + (STRIPE // K) * K, rem)])

    @pl.when(sid == NS - 1)
    def _():
        pltpu.sync_copy(rows0_v.at[pl.ds(0, TAIL)],
                        acc.at[pl.ds(NS * STRIPE, TAIL)])

    pltpu.sync_copy(s1_hbm, s1_v)
    pltpu.sync_copy(s2_hbm, s2_v)

    plsc.subcore_barrier()

    base0 = sid * EPS

    def start_idx(g, slot):
        pltpu.async_copy(
            idx_hbm.at[pl.ds((sid * NBLK + g) * 2 * K, 2 * K)],
            idx[slot], isem[slot])

    def wait_idx(slot):
        pltpu.make_async_copy(idx_hbm.at[pl.ds(0, 2 * K)], idx[slot],
                              isem[slot]).wait()

    def compute_attn(g, slot):
        for t in range(K // 16):
            sl = pl.ds(t * 16, 16)
            s16 = idx[slot][sl]
            d16 = idx[slot][pl.ds(K + t * 16, 16)]
            gidx[slot][sl] = d16 * 2 + cid
            srcc[slot][sl] = s16
            lg = (plsc.load_gather(s1_v, [s16])
                  + plsc.load_gather(s2_v, [d16]))
            lr = jnp.where(lg > 0.0, lg, 0.2 * lg)
            at = jnp.exp(lr)
            glob = base0 + g * K + t * 16 + lax.iota(jnp.int32, 16)
            at = jnp.where(glob < E, at, 0.0)
            attn[slot][sl] = at
            plsc.addupdate_scatter(asum_v, [s16], at)

    def start_gather(slot):
        pltpu.async_copy(embi_hbm.at[gidx[slot]], rows[slot], gsem[slot])

    def wait_gather(slot):
        pltpu.make_async_copy(embi_hbm.at[gidx[slot]], rows[slot],
                              gsem[slot]).wait()

    def scale_rows(slot):
        rv = rows[slot]
        av = attn[slot]

        @pl.loop(0, K // 4)
        def _(q):
            for u in range(4):
                r = q * 4 + u
                sp = plsc.load_gather(av, [jnp.full((16,), r, jnp.int32)])
                for ch in range(DH // 16):
                    cs = pl.ds(ch * 16, 16)
                    rv[r, cs] = rv[r, cs] * sp

    def start_scatter(slot):
        pltpu.async_copy(rows[slot], acc.at[srcc[slot]], csem[slot],
                         add=True)

    def wait_scatter(slot):
        pltpu.make_async_copy(rows[slot], acc.at[srcc[slot]],
                              csem[slot]).wait()

    def steady(g, slot):
        nslot = 1 - slot
        wait_scatter(nslot)
        wait_idx(nslot)
        compute_attn(g + 1, nslot)
        start_idx(g + 3, nslot)
        start_gather(nslot)
        wait_gather(slot)
        scale_rows(slot)
        start_scatter(slot)

    start_idx(0, 0)
    start_idx(1, 1)
    wait_idx(0)
    compute_attn(0, 0)
    start_idx(2, 0)
    start_gather(0)
    wait_idx(1)
    compute_attn(1, 1)
    start_idx(3, 1)
    start_gather(1)
    wait_gather(0)
    scale_rows(0)
    start_scatter(0)

    @pl.loop(0, (NBLK - 2) // 2)
    def _(t):
        steady(2 * t + 1, 1)
        steady(2 * t + 2, 0)

    wait_gather(1)
    scale_rows(1)
    start_scatter(1)
    wait_scatter(0)
    wait_scatter(1)
    wait_idx(0)
    wait_idx(1)

    @pl.when(cid == 0)
    def _():
        pltpu.sync_copy(asum_v, stage_hbm.at[sid])

    plsc.subcore_barrier()

    rs = pl.ds(sid * STRIPE, STRIPE)
    tl = pl.ds(NS * STRIPE, TAIL)
    last = sid == NS - 1

    @pl.when(cid == 0)
    def _():
        pltpu.sync_copy(acc.at[rs], aggl_hbm.at[rs])

        @pl.when(last)
        def _():
            pltpu.sync_copy(acc.at[tl], aggl_hbm.at[tl])

    @pl.when(cid == 1)
    def _():
        pltpu.sync_copy(acc.at[rs], aggr_hbm.at[rs])

        @pl.when(last)
        def _():
            pltpu.sync_copy(acc.at[tl], aggr_hbm.at[tl])

    @pl.when(cid == 0)
    def _():
        for j in range(RST // DH):
            n0 = sid * RST + j * DH

            @pl.when(n0 + DH <= NP)
            def _():
                pltpu.sync_copy(stage_hbm.at[:, pl.ds(n0, DH)],
                                rows0_v.at[pl.ds(0, NS)])
                for w in range(1, NS):
                    for ch in range(DH // 16):
                        cs = pl.ds(ch * 16, 16)
                        rows0_v[0, cs] = rows0_v[0, cs] + rows0_v[w, cs]
                pltpu.sync_copy(rows0_v.at[0], asum_hbm.at[pl.ds(n0, DH)])


_RB = 2000


def _post_body(emb_ref, aggl_ref, aggr_ref, asum_ref, tw_ref, tb_ref,
               g_ref, b_ref, out_ref):
    inv = 1.0 / (asum_ref[...] + 1e-20)
    dn = (((1,), (1,)), ((), ()))
    h = lax.dot_general(emb_ref[...], tw_ref[:, 0:D], dn,
                        preferred_element_type=jnp.float32)
    h = h + lax.dot_general(aggl_ref[...] * inv, tw_ref[:, D : D + DH], dn,
                            preferred_element_type=jnp.float32)
    h = h + lax.dot_general(aggr_ref[...] * inv, tw_ref[:, D + DH : 2 * D],
                            dn, preferred_element_type=jnp.float32)
    h = h + tb_ref[...][None, :]
    mean = jnp.mean(h, axis=1, keepdims=True)
    var = jnp.mean((h - mean) ** 2, axis=1, keepdims=True)
    out_ref[...] = ((h - mean) * lax.rsqrt(var + 1e-5) * g_ref[...][None, :]
                    + b_ref[...][None, :])


def _post_call(emb, aggl, aggr, asum, trans_weight, trans_bias,
               ln_gamma, ln_beta):
    nblk = N // _RB
    return pl.pallas_call(
        _post_body,
        grid=(nblk,),
        in_specs=[
            pl.BlockSpec((_RB, D), lambda i: (i, 0)),
            pl.BlockSpec((_RB, DH), lambda i: (i, 0)),
            pl.BlockSpec((_RB, DH), lambda i: (i, 0)),
            pl.BlockSpec((_RB, 1), lambda i: (i, 0)),
            pl.BlockSpec((D, 2 * D), lambda i: (0, 0)),
            pl.BlockSpec((D,), lambda i: (0,)),
            pl.BlockSpec((D,), lambda i: (0,)),
            pl.BlockSpec((D,), lambda i: (0,)),
        ],
        out_specs=pl.BlockSpec((_RB, D), lambda i: (i, 0)),
        out_shape=jax.ShapeDtypeStruct((N, D), jnp.float32),
    )(emb, aggl, aggr, asum[:N, None], trans_weight, trans_bias,
      ln_gamma, ln_beta)


def kernel(emb, edges, w_weight, w_bias, a_weight, a_bias,
           trans_weight, trans_bias, ln_gamma, ln_beta):
    src = edges[0]
    dst = edges[1]
    pad = EPAD - E
    srcp = jnp.concatenate([src, jnp.zeros((pad,), jnp.int32)])
    dstp = jnp.concatenate([dst, jnp.zeros((pad,), jnp.int32)])
    idxc = jnp.stack([srcp.reshape(-1, K), dstp.reshape(-1, K)],
                     axis=1).reshape(-1)
    idxc = jnp.concatenate([idxc, jnp.zeros((2 * 2 * K,), jnp.int32)])
    embi = emb.reshape(N, 2, DH).reshape(2 * N, DH)

    s1, s2 = _s12_call(emb, w_weight, w_bias, a_weight, a_bias)
    aggl, aggr, asum, _ = _sc_agg(embi, idxc, s1, s2)
    return _post_call(emb, aggl, aggr, asum, trans_weight, trans_bias,
                      ln_gamma, ln_beta)
